# Initial kernel scaffold; baseline (speedup 1.0000x reference)
import jax
import jax.numpy as jnp
from jax.experimental import pallas as pl


def kernel(x, edge_index, W1, b1, W2, b2):
    raise NotImplementedError("write your pallas kernel here")


# speedup vs baseline: 4.6820x; 4.6820x over previous
import functools

import jax
import jax.numpy as jnp
from jax import lax
from jax.experimental import pallas as pl
from jax.experimental.pallas import tpu as pltpu
from jax.experimental.pallas import tpu_sc as plsc

_NC = 2
_NS = 16
_NW = _NC * _NS


def _make_deg(n, e, k_chunk):
    epw = e // _NW
    iters = epw // k_chunk
    rpt = n // _NS
    mesh = plsc.VectorSubcoreMesh(core_axis_name="c", subcore_axis_name="s")

    @functools.partial(
        pl.kernel,
        mesh=mesh,
        compiler_params=pltpu.CompilerParams(use_tc_tiling_on_sc=False),
        out_type=[
            jax.ShapeDtypeStruct((_NC, n, 16), jnp.float32),
            jax.ShapeDtypeStruct((_NC, n, 16), jnp.float32),
        ],
        scratch_types=[
            pltpu.VMEM((k_chunk,), jnp.int32),
            pltpu.VMEM((k_chunk,), jnp.int32),
            pltpu.VMEM((k_chunk, 16), jnp.float32),
            pltpu.VMEM_SHARED((n, 16), jnp.float32),
            pltpu.VMEM_SHARED((n, 16), jnp.float32),
        ],
    )
    def deg(src_hbm, dst_hbm, zeros_hbm, ones_hbm, out_o_hbm, out_i_hbm,
            sidx, didx, ones_v, acc_o, acc_i):
        cid = lax.axis_index("c")
        sid = lax.axis_index("s")
        r0 = sid * rpt
        pltpu.sync_copy(zeros_hbm.at[pl.ds(r0, rpt)], acc_o.at[pl.ds(r0, rpt)])
        pltpu.sync_copy(zeros_hbm.at[pl.ds(r0, rpt)], acc_i.at[pl.ds(r0, rpt)])
        pltpu.sync_copy(ones_hbm, ones_v)
        plsc.subcore_barrier()
        wid = sid * _NC + cid
        base = wid * epw

        def body(i, carry):
            off = base + i * k_chunk
            pltpu.sync_copy(src_hbm.at[pl.ds(off, k_chunk)], sidx)
            pltpu.sync_copy(dst_hbm.at[pl.ds(off, k_chunk)], didx)
            pltpu.sync_copy(ones_v, acc_o.at[sidx], add=True)
            pltpu.sync_copy(ones_v, acc_i.at[didx], add=True)
            return carry

        lax.fori_loop(0, iters, body, 0)
        plsc.subcore_barrier()
        pltpu.sync_copy(acc_o.at[pl.ds(r0, rpt)], out_o_hbm.at[cid, pl.ds(r0, rpt)])
        pltpu.sync_copy(acc_i.at[pl.ds(r0, rpt)], out_i_hbm.at[cid, pl.ds(r0, rpt)])

    return deg


def _make_mp(n, d, e, k_chunk):
    epw = e // _NW
    iters = epw // k_chunk
    rpt = n // _NS
    mesh = plsc.VectorSubcoreMesh(core_axis_name="c", subcore_axis_name="s")

    @functools.partial(
        pl.kernel,
        mesh=mesh,
        out_type=jax.ShapeDtypeStruct((_NC, n, d), jnp.float32),
        scratch_types=[
            pltpu.VMEM((k_chunk,), jnp.int32),
            pltpu.VMEM((k_chunk,), jnp.int32),
            pltpu.VMEM((k_chunk, d), jnp.float32),
            pltpu.VMEM_SHARED((n, d), jnp.float32),
            pltpu.SemaphoreType.DMA,
        ],
    )
    def mp(x_hbm, src_hbm, dst_hbm, zeros_hbm, out_hbm,
           sidx, didx, rows, acc, sem):
        cid = lax.axis_index("c")
        sid = lax.axis_index("s")
        r0 = sid * rpt
        pltpu.sync_copy(zeros_hbm.at[pl.ds(r0, rpt)], acc.at[pl.ds(r0, rpt)])
        plsc.subcore_barrier()
        wid = sid * _NC + cid
        base = wid * epw

        def body(i, carry):
            off = base + i * k_chunk
            pltpu.sync_copy(src_hbm.at[pl.ds(off, k_chunk)], sidx)
            pltpu.sync_copy(dst_hbm.at[pl.ds(off, k_chunk)], didx)
            pltpu.async_copy(x_hbm.at[sidx], rows, sem).wait()
            pltpu.sync_copy(rows, acc.at[didx], add=True)
            return carry

        lax.fori_loop(0, iters, body, 0)
        plsc.subcore_barrier()
        pltpu.sync_copy(acc.at[pl.ds(r0, rpt)], out_hbm.at[cid, pl.ds(r0, rpt)])

    return mp


def _make_prep(n, d, blk):
    def body(dop_ref, dip_ref, x_ref, x1_ref, nin_ref, nout_ref):
        do = dop_ref[0, :, 0] + dop_ref[1, :, 0]
        di = dip_ref[0, :, 0] + dip_ref[1, :, 0]
        no = lax.rsqrt(jnp.maximum(do, 1.0))
        ni = lax.rsqrt(jnp.maximum(di, 1.0))
        x1_ref[...] = x_ref[...] * no[:, None]
        nin_ref[...] = jnp.broadcast_to(ni[:, None], (blk, d))
        nout_ref[...] = jnp.broadcast_to(no[:, None], (blk, d))

    return pl.pallas_call(
        body,
        grid=(n // blk,),
        in_specs=[
            pl.BlockSpec((2, blk, 16), lambda i: (0, i, 0)),
            pl.BlockSpec((2, blk, 16), lambda i: (0, i, 0)),
            pl.BlockSpec((blk, d), lambda i: (i, 0)),
        ],
        out_specs=[
            pl.BlockSpec((blk, d), lambda i: (i, 0)),
            pl.BlockSpec((blk, d), lambda i: (i, 0)),
            pl.BlockSpec((blk, d), lambda i: (i, 0)),
        ],
        out_shape=[jax.ShapeDtypeStruct((n, d), jnp.float32)] * 3,
    )


def _make_layer(n, d, blk, final):
    def body(ap_ref, nin_ref, nout_ref, w_ref, b_ref, o_ref):
        agg = (ap_ref[0] + ap_ref[1]) * nin_ref[...]
        h = jnp.dot(agg, w_ref[...], preferred_element_type=jnp.float32)
        h = h + b_ref[...]
        if final:
            nrm = jnp.sqrt(jnp.sum(h * h, axis=1, keepdims=True))
            o_ref[...] = h / nrm
        else:
            o_ref[...] = jnp.maximum(h, 0.0) * nout_ref[...]

    return pl.pallas_call(
        body,
        grid=(n // blk,),
        in_specs=[
            pl.BlockSpec((2, blk, d), lambda i: (0, i, 0)),
            pl.BlockSpec((blk, d), lambda i: (i, 0)),
            pl.BlockSpec((blk, d), lambda i: (i, 0)),
            pl.BlockSpec((d, d), lambda i: (0, 0)),
            pl.BlockSpec((1, d), lambda i: (0, 0)),
        ],
        out_specs=pl.BlockSpec((blk, d), lambda i: (i, 0)),
        out_shape=jax.ShapeDtypeStruct((n, d), jnp.float32),
    )


def kernel(x, edge_index, W1, b1, W2, b2):
    n, d = x.shape
    e = edge_index.shape[1]
    k_chunk = 80
    np_ = ((n + 2047) // 2048) * 2048
    blk = np_ // 10
    assert e % (_NW * k_chunk) == 0 and np_ % (_NS * 8) == 0 and blk % 8 == 0

    src = edge_index[0]
    dst = edge_index[1]
    xp = jnp.pad(x, ((0, np_ - n), (0, 0)))
    zeros_nd = jnp.zeros((np_, d), jnp.float32)
    zeros_n16 = jnp.zeros((np_, 16), jnp.float32)
    ones_k16 = jnp.ones((k_chunk, 16), jnp.float32)

    dop, dip = _make_deg(np_, e, k_chunk)(src, dst, zeros_n16, ones_k16)
    x1, nin, nout = _make_prep(np_, d, blk)(dop, dip, xp)
    mp = _make_mp(np_, d, e, k_chunk)
    agg1 = mp(x1, src, dst, zeros_nd)
    h1s = _make_layer(np_, d, blk, final=False)(
        agg1, nin, nout, W1, b1.reshape(1, d))
    agg2 = mp(h1s, src, dst, zeros_nd)
    out = _make_layer(np_, d, blk, final=True)(
        agg2, nin, nout, W2, b2.reshape(1, d))
    return out[:n]


# speedup vs baseline: 4.9879x; 1.0653x over previous
import functools

import jax
import jax.numpy as jnp
from jax import lax
from jax.experimental import pallas as pl
from jax.experimental.pallas import tpu as pltpu
from jax.experimental.pallas import tpu_sc as plsc

_NC = 2
_NS = 16
_NW = _NC * _NS
_K = 128


def _zero_fill(ref, nrow, ncol):
    z = jnp.zeros((16,), jnp.float32)

    def body(t, carry):
        r = t // (ncol // 16)
        c = t % (ncol // 16)
        ref[r, pl.ds(c * 16, 16)] = z
        return carry

    lax.fori_loop(0, nrow * (ncol // 16), body, 0)


def _make_deg(n, iters):
    rpt = n // _NS
    mesh = plsc.VectorSubcoreMesh(core_axis_name="c", subcore_axis_name="s")

    @functools.partial(
        pl.kernel,
        mesh=mesh,
        compiler_params=pltpu.CompilerParams(use_tc_tiling_on_sc=False),
        out_type=[
            jax.ShapeDtypeStruct((_NC, n, 16), jnp.float32),
            jax.ShapeDtypeStruct((_NC, n, 16), jnp.float32),
        ],
        scratch_types=[
            pltpu.VMEM((iters, _K), jnp.int32),
            pltpu.VMEM((iters, _K), jnp.int32),
            pltpu.VMEM((_K, 16), jnp.float32),
            pltpu.VMEM((_K, 16), jnp.float32),
            pltpu.VMEM_SHARED((n, 16), jnp.float32),
            pltpu.VMEM_SHARED((n, 16), jnp.float32),
            pltpu.SemaphoreType.DMA,
            pltpu.SemaphoreType.DMA,
        ],
    )
    def deg(src_hbm, dst_hbm, out_o_hbm, out_i_hbm,
            sidx, didx, ones_v, zbuf, acc_o, acc_i, sema, semb):
        cid = lax.axis_index("c")
        sid = lax.axis_index("s")
        r0 = sid * rpt
        wid = sid * _NC + cid
        pltpu.sync_copy(src_hbm.at[wid], sidx)
        pltpu.sync_copy(dst_hbm.at[wid], didx)
        _zero_fill(zbuf, _K, 16)
        o = jnp.ones((16,), jnp.float32)

        def obody(t, carry):
            ones_v[t, pl.ds(0, 16)] = o
            return carry

        lax.fori_loop(0, _K, obody, 0)
        for t in range(rpt // _K):
            pltpu.sync_copy(zbuf, acc_o.at[pl.ds(r0 + t * _K, _K)])
            pltpu.sync_copy(zbuf, acc_i.at[pl.ds(r0 + t * _K, _K)])
        plsc.subcore_barrier()

        def fire(i):
            pltpu.async_copy(ones_v, acc_o.at[sidx.at[i]], sema, add=True)
            pltpu.async_copy(ones_v, acc_i.at[didx.at[i]], semb, add=True)

        def drain(i):
            pltpu.make_async_copy(ones_v, acc_o.at[sidx.at[i]], sema).wait()
            pltpu.make_async_copy(ones_v, acc_i.at[didx.at[i]], semb).wait()

        fire(0)

        def body(i, carry):
            fire(i)
            drain(i - 1)
            return carry

        lax.fori_loop(1, iters, body, 0)
        drain(iters - 1)
        plsc.subcore_barrier()
        pltpu.sync_copy(acc_o.at[pl.ds(r0, rpt)], out_o_hbm.at[cid, pl.ds(r0, rpt)])
        pltpu.sync_copy(acc_i.at[pl.ds(r0, rpt)], out_i_hbm.at[cid, pl.ds(r0, rpt)])

    return deg


def _make_mp(n, d, iters):
    rpt = n // _NS
    mesh = plsc.VectorSubcoreMesh(core_axis_name="c", subcore_axis_name="s")

    @functools.partial(
        pl.kernel,
        mesh=mesh,
        out_type=jax.ShapeDtypeStruct((_NC, n, d), jnp.float32),
        scratch_types=[
            pltpu.VMEM((3, 2, _K), jnp.int32),
            pltpu.VMEM((2, _K, d), jnp.float32),
            pltpu.VMEM_SHARED((n, d), jnp.float32),
            pltpu.SemaphoreType.DMA,
            pltpu.SemaphoreType.DMA,
        ],
    )
    def mp(x_hbm, ei_hbm, out_hbm, idx, rows, acc, isem, gsem):
        cid = lax.axis_index("c")
        sid = lax.axis_index("s")
        r0 = sid * rpt
        wid = sid * _NC + cid

        def iload(i):
            pltpu.async_copy(ei_hbm.at[wid, i], idx.at[i % 3], isem)

        def iwait(i):
            pltpu.make_async_copy(ei_hbm.at[wid, i], idx.at[i % 3], isem).wait()

        def gfire(i):
            pltpu.async_copy(x_hbm.at[idx.at[i % 3, 0]], rows.at[i % 2], gsem)

        def gwait(i):
            pltpu.make_async_copy(
                x_hbm.at[idx.at[i % 3, 0]], rows.at[i % 2], gsem).wait()

        iload(0)
        iload(1)
        iwait(0)
        gfire(0)
        _zero_fill(rows.at[1], _K, d)
        for t in range(rpt // _K):
            pltpu.sync_copy(rows.at[1], acc.at[pl.ds(r0 + t * _K, _K)])
        plsc.subcore_barrier()

        def body(i, carry):
            @pl.when(i + 2 < iters)
            def _():
                iload(i + 2)

            gwait(i)
            pltpu.sync_copy(rows.at[i % 2], acc.at[idx.at[i % 3, 1]], add=True)

            @pl.when(i + 1 < iters)
            def _():
                iwait(i + 1)
                gfire(i + 1)

            return carry

        lax.fori_loop(0, iters, body, 0)
        plsc.subcore_barrier()
        pltpu.sync_copy(acc.at[pl.ds(r0, rpt)], out_hbm.at[cid, pl.ds(r0, rpt)])

    return mp


def _make_prep(n, d, blk):
    def body(dop_ref, dip_ref, x_ref, x1_ref, nin_ref, nout_ref):
        do = dop_ref[0, :, 0] + dop_ref[1, :, 0]
        di = dip_ref[0, :, 0] + dip_ref[1, :, 0]
        no = lax.rsqrt(jnp.maximum(do, 1.0))
        ni = lax.rsqrt(jnp.maximum(di, 1.0))
        x1_ref[...] = x_ref[...] * no[:, None]
        nin_ref[...] = jnp.broadcast_to(ni[:, None], (blk, d))
        nout_ref[...] = jnp.broadcast_to(no[:, None], (blk, d))

    return pl.pallas_call(
        body,
        grid=(n // blk,),
        in_specs=[
            pl.BlockSpec((2, blk, 16), lambda i: (0, i, 0)),
            pl.BlockSpec((2, blk, 16), lambda i: (0, i, 0)),
            pl.BlockSpec((blk, d), lambda i: (i, 0)),
        ],
        out_specs=[
            pl.BlockSpec((blk, d), lambda i: (i, 0)),
            pl.BlockSpec((blk, d), lambda i: (i, 0)),
            pl.BlockSpec((blk, d), lambda i: (i, 0)),
        ],
        out_shape=[jax.ShapeDtypeStruct((n, d), jnp.float32)] * 3,
    )


def _make_layer(n, d, blk, final):
    def body(ap_ref, nin_ref, nout_ref, w_ref, b_ref, o_ref):
        agg = (ap_ref[0] + ap_ref[1]) * nin_ref[...]
        h = jnp.dot(agg, w_ref[...], preferred_element_type=jnp.float32)
        h = h + b_ref[...]
        if final:
            nrm = jnp.sqrt(jnp.sum(h * h, axis=1, keepdims=True))
            o_ref[...] = h / nrm
        else:
            o_ref[...] = jnp.maximum(h, 0.0) * nout_ref[...]

    return pl.pallas_call(
        body,
        grid=(n // blk,),
        in_specs=[
            pl.BlockSpec((2, blk, d), lambda i: (0, i, 0)),
            pl.BlockSpec((blk, d), lambda i: (i, 0)),
            pl.BlockSpec((blk, d), lambda i: (i, 0)),
            pl.BlockSpec((d, d), lambda i: (0, 0)),
            pl.BlockSpec((1, d), lambda i: (0, 0)),
        ],
        out_specs=pl.BlockSpec((blk, d), lambda i: (i, 0)),
        out_shape=jax.ShapeDtypeStruct((n, d), jnp.float32),
    )


def kernel(x, edge_index, W1, b1, W2, b2):
    n, d = x.shape
    e = edge_index.shape[1]
    np_ = ((n + 2047) // 2048) * 2048
    blk = np_ // 10
    per_w = -(-e // _NW)
    iters = -(-per_w // _K)
    ep = _NW * iters * _K
    assert np_ % (_NS * _K) == 0 and blk % 8 == 0

    src = jnp.pad(edge_index[0], (0, ep - e), constant_values=n)
    dst = jnp.pad(edge_index[1], (0, ep - e), constant_values=n)
    src3 = src.reshape(_NW, iters, _K)
    dst3 = dst.reshape(_NW, iters, _K)
    ei4 = jnp.stack([src3, dst3], axis=2)
    xp = jnp.pad(x, ((0, np_ - n), (0, 0)))

    dop, dip = _make_deg(np_, iters)(src3, dst3)
    x1, nin, nout = _make_prep(np_, d, blk)(dop, dip, xp)
    mp = _make_mp(np_, d, iters)
    agg1 = mp(x1, ei4)
    h1s = _make_layer(np_, d, blk, final=False)(
        agg1, nin, nout, W1, b1.reshape(1, d))
    agg2 = mp(h1s, ei4)
    out = _make_layer(np_, d, blk, final=True)(
        agg2, nin, nout, W2, b2.reshape(1, d))
    return out[:n]


# speedup vs baseline: 6.9920x; 1.4018x over previous
import functools

import jax
import jax.numpy as jnp
from jax import lax
from jax.experimental import pallas as pl
from jax.experimental.pallas import tpu as pltpu
from jax.experimental.pallas import tpu_sc as plsc

_NC = 2
_NS = 16
_NW = _NC * _NS
_K = 128


def _make_deg(n, iters):
    rpt = n // _NS
    mesh = plsc.VectorSubcoreMesh(core_axis_name="c", subcore_axis_name="s")

    @functools.partial(
        pl.kernel,
        mesh=mesh,
        compiler_params=pltpu.CompilerParams(use_tc_tiling_on_sc=False),
        out_type=[
            jax.ShapeDtypeStruct((_NC, n, 16), jnp.float32),
            jax.ShapeDtypeStruct((_NC, n, 16), jnp.float32),
        ],
        scratch_types=[
            pltpu.VMEM((iters, _K), jnp.int32),
            pltpu.VMEM((iters, _K), jnp.int32),
            pltpu.VMEM((_K, 16), jnp.float32),
            pltpu.VMEM_SHARED((n, 16), jnp.float32),
            pltpu.VMEM_SHARED((n, 16), jnp.float32),
            pltpu.SemaphoreType.DMA,
            pltpu.SemaphoreType.DMA,
        ],
    )
    def deg(src_hbm, dst_hbm, ones_hbm, zeros_hbm, out_o_hbm, out_i_hbm,
            sidx, didx, ones_v, acc_o, acc_i, sema, semb):
        cid = lax.axis_index("c")
        sid = lax.axis_index("s")
        r0 = sid * rpt
        wid = sid * _NC + cid
        pltpu.sync_copy(src_hbm.at[wid], sidx)
        pltpu.sync_copy(dst_hbm.at[wid], didx)
        pltpu.sync_copy(ones_hbm, ones_v)
        pltpu.sync_copy(zeros_hbm.at[pl.ds(r0, rpt)], acc_o.at[pl.ds(r0, rpt)])
        pltpu.sync_copy(zeros_hbm.at[pl.ds(r0, rpt)], acc_i.at[pl.ds(r0, rpt)])
        plsc.subcore_barrier()

        def fire(i):
            pltpu.async_copy(ones_v, acc_o.at[sidx.at[i]], sema, add=True)
            pltpu.async_copy(ones_v, acc_i.at[didx.at[i]], semb, add=True)

        def drain(i):
            pltpu.make_async_copy(ones_v, acc_o.at[sidx.at[i]], sema).wait()
            pltpu.make_async_copy(ones_v, acc_i.at[didx.at[i]], semb).wait()

        fire(0)

        def body(i, carry):
            fire(i)
            drain(i - 1)
            return carry

        lax.fori_loop(1, iters, body, 0)
        drain(iters - 1)
        plsc.subcore_barrier()
        pltpu.sync_copy(acc_o.at[pl.ds(r0, rpt)], out_o_hbm.at[cid, pl.ds(r0, rpt)])
        pltpu.sync_copy(acc_i.at[pl.ds(r0, rpt)], out_i_hbm.at[cid, pl.ds(r0, rpt)])

    return deg


def _make_mp(n, dh, iters):
    rpt = n // _NS
    mesh = plsc.VectorSubcoreMesh(core_axis_name="c", subcore_axis_name="s")

    @functools.partial(
        pl.kernel,
        mesh=mesh,
        compiler_params=pltpu.CompilerParams(use_tc_tiling_on_sc=False),
        out_type=jax.ShapeDtypeStruct((_NC, n, dh), jnp.float32),
        scratch_types=[
            pltpu.VMEM((3, 2, _K), jnp.int32),
            pltpu.VMEM((2, _K, dh), jnp.float32),
            pltpu.VMEM_SHARED((n, dh), jnp.float32),
            pltpu.VMEM_SHARED((n, dh), jnp.float32),
            pltpu.SemaphoreType.DMA,
            pltpu.SemaphoreType.DMA,
        ],
    )
    def mp(x_hbm, ei_hbm, zeros_hbm, out_hbm, idx, rows, xsp, acc, isem, gsem):
        cid = lax.axis_index("c")
        sid = lax.axis_index("s")
        r0 = sid * rpt

        def iload(i):
            pltpu.async_copy(ei_hbm.at[sid, i], idx.at[i % 3], isem)

        def iwait(i):
            pltpu.make_async_copy(ei_hbm.at[sid, i], idx.at[i % 3], isem).wait()

        def gfire(i):
            pltpu.async_copy(xsp.at[idx.at[i % 3, 0]], rows.at[i % 2], gsem)

        def gwait(i):
            pltpu.make_async_copy(
                xsp.at[idx.at[i % 3, 0]], rows.at[i % 2], gsem).wait()

        iload(0)
        iload(1)
        pltpu.sync_copy(x_hbm.at[cid, pl.ds(r0, rpt)], xsp.at[pl.ds(r0, rpt)])
        pltpu.sync_copy(zeros_hbm.at[pl.ds(r0, rpt)], acc.at[pl.ds(r0, rpt)])
        plsc.subcore_barrier()
        iwait(0)
        gfire(0)

        def body(i, carry):
            gwait(i)
            pltpu.sync_copy(rows.at[i % 2], acc.at[idx.at[i % 3, 1]], add=True)

            @pl.when(i + 2 < iters)
            def _():
                iload(i + 2)

            @pl.when(i + 1 < iters)
            def _():
                iwait(i + 1)
                gfire(i + 1)

            return carry

        lax.fori_loop(0, iters, body, 0)
        plsc.subcore_barrier()
        pltpu.sync_copy(acc.at[pl.ds(r0, rpt)], out_hbm.at[cid, pl.ds(r0, rpt)])

    return mp


def _make_prep(n, d, blk):
    dh = d // 2

    def body(dop_ref, dip_ref, x_ref, x1_ref, nin_ref, nout_ref):
        do = dop_ref[0, :, 0] + dop_ref[1, :, 0]
        di = dip_ref[0, :, 0] + dip_ref[1, :, 0]
        no = lax.rsqrt(jnp.maximum(do, 1.0))
        ni = lax.rsqrt(jnp.maximum(di, 1.0))
        x1 = x_ref[...] * no[:, None]
        x1_ref[0] = x1[:, :dh]
        x1_ref[1] = x1[:, dh:]
        nin_ref[...] = jnp.broadcast_to(ni[:, None], (blk, d))
        nout_ref[...] = jnp.broadcast_to(no[:, None], (blk, d))

    return pl.pallas_call(
        body,
        grid=(n // blk,),
        in_specs=[
            pl.BlockSpec((2, blk, 16), lambda i: (0, i, 0)),
            pl.BlockSpec((2, blk, 16), lambda i: (0, i, 0)),
            pl.BlockSpec((blk, d), lambda i: (i, 0)),
        ],
        out_specs=[
            pl.BlockSpec((2, blk, dh), lambda i: (0, i, 0)),
            pl.BlockSpec((blk, d), lambda i: (i, 0)),
            pl.BlockSpec((blk, d), lambda i: (i, 0)),
        ],
        out_shape=[
            jax.ShapeDtypeStruct((2, n, dh), jnp.float32),
            jax.ShapeDtypeStruct((n, d), jnp.float32),
            jax.ShapeDtypeStruct((n, d), jnp.float32),
        ],
    )


def _make_layer(n, d, blk, final):
    dh = d // 2

    def body(ap_ref, nin_ref, nout_ref, w_ref, b_ref, o_ref):
        agg = jnp.concatenate([ap_ref[0], ap_ref[1]], axis=1) * nin_ref[...]
        h = jnp.dot(agg, w_ref[...], preferred_element_type=jnp.float32)
        h = h + b_ref[...]
        if final:
            nrm = jnp.sqrt(jnp.sum(h * h, axis=1, keepdims=True))
            o_ref[...] = h / nrm
        else:
            h = jnp.maximum(h, 0.0) * nout_ref[...]
            o_ref[0] = h[:, :dh]
            o_ref[1] = h[:, dh:]

    if final:
        out_spec = pl.BlockSpec((blk, d), lambda i: (i, 0))
        out_shape = jax.ShapeDtypeStruct((n, d), jnp.float32)
    else:
        out_spec = pl.BlockSpec((2, blk, dh), lambda i: (0, i, 0))
        out_shape = jax.ShapeDtypeStruct((2, n, dh), jnp.float32)

    return pl.pallas_call(
        body,
        grid=(n // blk,),
        in_specs=[
            pl.BlockSpec((2, blk, dh), lambda i: (0, i, 0)),
            pl.BlockSpec((blk, d), lambda i: (i, 0)),
            pl.BlockSpec((blk, d), lambda i: (i, 0)),
            pl.BlockSpec((d, d), lambda i: (0, 0)),
            pl.BlockSpec((1, d), lambda i: (0, 0)),
        ],
        out_specs=out_spec,
        out_shape=out_shape,
    )


def kernel(x, edge_index, W1, b1, W2, b2):
    n, d = x.shape
    dh = d // 2
    e = edge_index.shape[1]
    np_ = ((n + 2047) // 2048) * 2048
    blk = np_ // 10
    assert np_ % (_NS * _K) == 0 and blk % 8 == 0

    it32 = -(-e // (_NW * _K))
    ep32 = _NW * it32 * _K
    src_p = jnp.pad(edge_index[0], (0, ep32 - e), constant_values=n)
    dst_p = jnp.pad(edge_index[1], (0, ep32 - e), constant_values=n)
    src3 = src_p.reshape(_NW, it32, _K)
    dst3 = dst_p.reshape(_NW, it32, _K)

    it16 = -(-e // (_NS * _K))
    ep16 = _NS * it16 * _K
    src_q = jnp.pad(edge_index[0], (0, ep16 - e), constant_values=n)
    dst_q = jnp.pad(edge_index[1], (0, ep16 - e), constant_values=n)
    ei4 = jnp.stack(
        [src_q.reshape(_NS, it16, _K), dst_q.reshape(_NS, it16, _K)], axis=2)

    xp = jnp.pad(x, ((0, np_ - n), (0, 0)))
    ones_k16 = jnp.ones((_K, 16), jnp.float32)
    zeros_n16 = jnp.zeros((np_, 16), jnp.float32)
    zeros_ndh = jnp.zeros((np_, dh), jnp.float32)

    dop, dip = _make_deg(np_, it32)(src3, dst3, ones_k16, zeros_n16)
    x1h, nin, nout = _make_prep(np_, d, blk)(dop, dip, xp)
    mp = _make_mp(np_, dh, it16)
    agg1 = mp(x1h, ei4, zeros_ndh)
    h1h = _make_layer(np_, d, blk, final=False)(
        agg1, nin, nout, W1, b1.reshape(1, d))
    agg2 = mp(h1h, ei4, zeros_ndh)
    out = _make_layer(np_, d, blk, final=True)(
        agg2, nin, nout, W2, b2.reshape(1, d))
    return out[:n]


# speedup vs baseline: 7.0672x; 1.0108x over previous
import functools

import jax
import jax.numpy as jnp
from jax import lax
from jax.experimental import pallas as pl
from jax.experimental.pallas import tpu as pltpu
from jax.experimental.pallas import tpu_sc as plsc

_NC = 2
_NS = 16
_NW = _NC * _NS
_K = 128


def _make_deg(n, iters):
    rpt = n // _NS
    mesh = plsc.VectorSubcoreMesh(core_axis_name="c", subcore_axis_name="s")

    @functools.partial(
        pl.kernel,
        mesh=mesh,
        compiler_params=pltpu.CompilerParams(use_tc_tiling_on_sc=False),
        out_type=[
            jax.ShapeDtypeStruct((_NC, n, 16), jnp.float32),
            jax.ShapeDtypeStruct((_NC, n, 16), jnp.float32),
        ],
        scratch_types=[
            pltpu.VMEM((3, 2, _K), jnp.int32),
            pltpu.VMEM((_K, 16), jnp.float32),
            pltpu.VMEM_SHARED((n, 16), jnp.float32),
            pltpu.VMEM_SHARED((n, 16), jnp.float32),
            pltpu.SemaphoreType.DMA,
            pltpu.SemaphoreType.DMA,
            pltpu.SemaphoreType.DMA,
        ],
    )
    def deg(ei_hbm, ones_hbm, zeros_hbm, out_o_hbm, out_i_hbm,
            idx, ones_v, acc_o, acc_i, isem, sema, semb):
        cid = lax.axis_index("c")
        sid = lax.axis_index("s")
        r0 = sid * rpt
        wid = sid * _NC + cid
        base = wid * iters * _K

        def iload(i):
            off = base + i * _K
            pltpu.async_copy(ei_hbm.at[0, pl.ds(off, _K)], idx.at[i % 3, 0], isem)
            pltpu.async_copy(ei_hbm.at[1, pl.ds(off, _K)], idx.at[i % 3, 1], isem)

        def iwait(i):
            off = base + i * _K
            pltpu.make_async_copy(
                ei_hbm.at[0, pl.ds(off, _K)], idx.at[i % 3, 0], isem).wait()
            pltpu.make_async_copy(
                ei_hbm.at[1, pl.ds(off, _K)], idx.at[i % 3, 1], isem).wait()

        def fire(i):
            pltpu.async_copy(ones_v, acc_o.at[idx.at[i % 3, 0]], sema, add=True)
            pltpu.async_copy(ones_v, acc_i.at[idx.at[i % 3, 1]], semb, add=True)

        def drain(i):
            pltpu.make_async_copy(ones_v, acc_o.at[idx.at[i % 3, 0]], sema).wait()
            pltpu.make_async_copy(ones_v, acc_i.at[idx.at[i % 3, 1]], semb).wait()

        iload(0)
        iload(1)
        pltpu.sync_copy(ones_hbm, ones_v)
        pltpu.sync_copy(zeros_hbm.at[pl.ds(r0, rpt)], acc_o.at[pl.ds(r0, rpt)])
        pltpu.sync_copy(zeros_hbm.at[pl.ds(r0, rpt)], acc_i.at[pl.ds(r0, rpt)])
        plsc.subcore_barrier()
        iwait(0)
        fire(0)

        def body(i, carry):
            @pl.when(i + 1 < iters)
            def _():
                iload(i + 1)

            iwait(i)
            fire(i)
            drain(i - 1)
            return carry

        lax.fori_loop(1, iters, body, 0)
        drain(iters - 1)
        plsc.subcore_barrier()
        pltpu.sync_copy(acc_o.at[pl.ds(r0, rpt)], out_o_hbm.at[cid, pl.ds(r0, rpt)])
        pltpu.sync_copy(acc_i.at[pl.ds(r0, rpt)], out_i_hbm.at[cid, pl.ds(r0, rpt)])

    return deg


def _make_mp(n, dh, iters):
    rpt = n // _NS
    mesh = plsc.VectorSubcoreMesh(core_axis_name="c", subcore_axis_name="s")

    @functools.partial(
        pl.kernel,
        mesh=mesh,
        compiler_params=pltpu.CompilerParams(use_tc_tiling_on_sc=False),
        out_type=jax.ShapeDtypeStruct((_NC, n, dh), jnp.float32),
        scratch_types=[
            pltpu.VMEM((3, 2, _K), jnp.int32),
            pltpu.VMEM((2, _K, dh), jnp.float32),
            pltpu.VMEM_SHARED((n, dh), jnp.float32),
            pltpu.VMEM_SHARED((n, dh), jnp.float32),
            pltpu.SemaphoreType.DMA,
            pltpu.SemaphoreType.DMA,
        ],
    )
    def mp(x_hbm, ei_hbm, zeros_hbm, out_hbm, idx, rows, xsp, acc, isem, gsem):
        cid = lax.axis_index("c")
        sid = lax.axis_index("s")
        r0 = sid * rpt
        base = sid * iters * _K

        def iload(i):
            off = base + i * _K
            pltpu.async_copy(ei_hbm.at[0, pl.ds(off, _K)], idx.at[i % 3, 0], isem)
            pltpu.async_copy(ei_hbm.at[1, pl.ds(off, _K)], idx.at[i % 3, 1], isem)

        def iwait(i):
            off = base + i * _K
            pltpu.make_async_copy(
                ei_hbm.at[0, pl.ds(off, _K)], idx.at[i % 3, 0], isem).wait()
            pltpu.make_async_copy(
                ei_hbm.at[1, pl.ds(off, _K)], idx.at[i % 3, 1], isem).wait()

        def gfire(i):
            pltpu.async_copy(xsp.at[idx.at[i % 3, 0]], rows.at[i % 2], gsem)

        def gwait(i):
            pltpu.make_async_copy(
                xsp.at[idx.at[i % 3, 0]], rows.at[i % 2], gsem).wait()

        iload(0)
        iload(1)
        pltpu.sync_copy(x_hbm.at[cid, pl.ds(r0, rpt)], xsp.at[pl.ds(r0, rpt)])
        pltpu.sync_copy(zeros_hbm.at[pl.ds(r0, rpt)], acc.at[pl.ds(r0, rpt)])
        plsc.subcore_barrier()
        iwait(0)
        gfire(0)

        def body(i, carry):
            gwait(i)
            pltpu.sync_copy(rows.at[i % 2], acc.at[idx.at[i % 3, 1]], add=True)

            @pl.when(i + 2 < iters)
            def _():
                iload(i + 2)

            @pl.when(i + 1 < iters)
            def _():
                iwait(i + 1)
                gfire(i + 1)

            return carry

        lax.fori_loop(0, iters, body, 0)
        plsc.subcore_barrier()
        pltpu.sync_copy(acc.at[pl.ds(r0, rpt)], out_hbm.at[cid, pl.ds(r0, rpt)])

    return mp


def _make_prep(n, d, blk):
    dh = d // 2

    def body(dop_ref, dip_ref, x_ref, x1_ref, nin_ref, nout_ref):
        do = dop_ref[0, :, 0] + dop_ref[1, :, 0]
        di = dip_ref[0, :, 0] + dip_ref[1, :, 0]
        no = lax.rsqrt(jnp.maximum(do, 1.0))
        ni = lax.rsqrt(jnp.maximum(di, 1.0))
        x1 = x_ref[...] * no[:, None]
        x1_ref[0] = x1[:, :dh]
        x1_ref[1] = x1[:, dh:]
        nin_ref[...] = jnp.broadcast_to(ni[:, None], (blk, d))
        nout_ref[...] = jnp.broadcast_to(no[:, None], (blk, d))

    return pl.pallas_call(
        body,
        grid=(n // blk,),
        in_specs=[
            pl.BlockSpec((2, blk, 16), lambda i: (0, i, 0)),
            pl.BlockSpec((2, blk, 16), lambda i: (0, i, 0)),
            pl.BlockSpec((blk, d), lambda i: (i, 0)),
        ],
        out_specs=[
            pl.BlockSpec((2, blk, dh), lambda i: (0, i, 0)),
            pl.BlockSpec((blk, d), lambda i: (i, 0)),
            pl.BlockSpec((blk, d), lambda i: (i, 0)),
        ],
        out_shape=[
            jax.ShapeDtypeStruct((2, n, dh), jnp.float32),
            jax.ShapeDtypeStruct((n, d), jnp.float32),
            jax.ShapeDtypeStruct((n, d), jnp.float32),
        ],
    )


def _make_layer(n, d, blk, final):
    dh = d // 2

    def body(ap_ref, nin_ref, nout_ref, w_ref, b_ref, o_ref):
        agg = jnp.concatenate([ap_ref[0], ap_ref[1]], axis=1) * nin_ref[...]
        h = jnp.dot(agg, w_ref[...], preferred_element_type=jnp.float32)
        h = h + b_ref[...]
        if final:
            nrm = jnp.sqrt(jnp.sum(h * h, axis=1, keepdims=True))
            o_ref[...] = h / nrm
        else:
            h = jnp.maximum(h, 0.0) * nout_ref[...]
            o_ref[0] = h[:, :dh]
            o_ref[1] = h[:, dh:]

    if final:
        out_spec = pl.BlockSpec((blk, d), lambda i: (i, 0))
        out_shape = jax.ShapeDtypeStruct((n, d), jnp.float32)
    else:
        out_spec = pl.BlockSpec((2, blk, dh), lambda i: (0, i, 0))
        out_shape = jax.ShapeDtypeStruct((2, n, dh), jnp.float32)

    return pl.pallas_call(
        body,
        grid=(n // blk,),
        in_specs=[
            pl.BlockSpec((2, blk, dh), lambda i: (0, i, 0)),
            pl.BlockSpec((blk, d), lambda i: (i, 0)),
            pl.BlockSpec((blk, d), lambda i: (i, 0)),
            pl.BlockSpec((d, d), lambda i: (0, 0)),
            pl.BlockSpec((1, d), lambda i: (0, 0)),
        ],
        out_specs=out_spec,
        out_shape=out_shape,
    )


def kernel(x, edge_index, W1, b1, W2, b2):
    n, d = x.shape
    dh = d // 2
    e = edge_index.shape[1]
    np_ = ((n + 2047) // 2048) * 2048
    blk = np_ // 10
    assert np_ % (_NS * _K) == 0 and blk % 8 == 0

    it32 = -(-e // (_NW * _K))
    it16 = 2 * it32
    ep = _NW * it32 * _K
    ei2 = jnp.pad(edge_index, ((0, 0), (0, ep - e)), constant_values=n)

    xp = jnp.pad(x, ((0, np_ - n), (0, 0)))
    ones_k16 = jnp.ones((_K, 16), jnp.float32)
    zeros_n16 = jnp.zeros((np_, 16), jnp.float32)
    zeros_ndh = jnp.zeros((np_, dh), jnp.float32)

    dop, dip = _make_deg(np_, it32)(ei2, ones_k16, zeros_n16)
    x1h, nin, nout = _make_prep(np_, d, blk)(dop, dip, xp)
    mp = _make_mp(np_, dh, it16)
    agg1 = mp(x1h, ei2, zeros_ndh)
    h1h = _make_layer(np_, d, blk, final=False)(
        agg1, nin, nout, W1, b1.reshape(1, d))
    agg2 = mp(h1h, ei2, zeros_ndh)
    out = _make_layer(np_, d, blk, final=True)(
        agg2, nin, nout, W2, b2.reshape(1, d))
    return out[:n]
